# Initial kernel scaffold; baseline (speedup 1.0000x reference)
import jax
import jax.numpy as jnp
from jax.experimental import pallas as pl


def kernel(index_tensor, ranking_tensor, positive_positions, weight):
    raise NotImplementedError("write your pallas kernel here")


# speedup vs baseline: 247.6489x; 247.6489x over previous
import functools

import jax
import jax.numpy as jnp
from jax import lax
from jax.experimental import pallas as pl
from jax.experimental.pallas import tpu as pltpu
from jax.experimental.pallas import tpu_sc as plsc

N_DOCS = 100000
RRF_K = 60.0
B, T, K = 256, 8, 100
E = T * K
NV = E // 16
EX = E + 16
NC, NS = 2, 16
NW = NC * NS
QPW = B // NW
ROUNDS = 5
BIG = N_DOCS


def _body(idx_hbm, rank_hbm, pp_hbm, w_hbm, out_hbm,
          idx_v, rank_v, w_v, pp_v, table_v, sel_v, docs_v, ans_v):
    wid = lax.axis_index("s") * NC + lax.axis_index("c")
    lane = lax.iota(jnp.int32, 16)
    zero16 = jnp.zeros((16,), jnp.float32)

    pltpu.sync_copy(idx_hbm.at[pl.ds(wid * (QPW * E), QPW * E)], idx_v)
    pltpu.sync_copy(rank_hbm.at[pl.ds(wid * (QPW * E), QPW * E)], rank_v)
    pltpu.sync_copy(w_hbm, w_v)
    pltpu.sync_copy(pp_hbm.at[pl.ds(wid * QPW, 16)], pp_v)

    pp_vec = pp_v[...]
    docs_v[pl.ds(E, 16)] = lane

    def one_query(q, ans_vec):
        base = q * E

        def p0(j, _):
            dv = idx_v[pl.ds(base + j * 16, 16)]
            docs_v[pl.ds(j * 16, 16)] = dv
            plsc.store_scatter(table_v, [dv], zero16)
            return 0
        lax.fori_loop(0, NV, p0, 0)
        plsc.store_scatter(table_v, [lane], zero16)

        def p1(j, _):
            dv = docs_v[pl.ds(j * 16, 16)]
            rv = rank_v[pl.ds(base + j * 16, 16)]
            wv = w_v[pl.ds(j * 16, 16)]
            plsc.addupdate_scatter(table_v, [dv], wv / (RRF_K + rv))
            return 0
        lax.fori_loop(0, NV, p1, 0)

        def p2(j, _):
            dv = docs_v[pl.ds(j * 16, 16)]
            sel_v[pl.ds(j * 16, 16)] = plsc.load_gather(table_v, [dv])
            return 0
        lax.fori_loop(0, NV + 1, p2, 0)

        def p3(j, _):
            dv = docs_v[pl.ds(j * 16, 16)]
            ev = j * 16 + lane
            plsc.store_scatter(table_v, [dv], plsc.bitcast(ev, jnp.float32))
            return 0
        lax.fori_loop(0, NV + 1, p3, 0)

        def p4(j, _):
            dv = docs_v[pl.ds(j * 16, 16)]
            tag = plsc.bitcast(plsc.load_gather(table_v, [dv]), jnp.int32)
            canon = tag == (j * 16 + lane)
            sel_v[pl.ds(j * 16, 16)] = jnp.where(
                canon, sel_v[pl.ds(j * 16, 16)], jnp.float32(-1.0))
            return 0
        lax.fori_loop(0, NV + 1, p4, 0)

        def one_round(r, ans_vec):
            def amax(j, acc):
                return jnp.maximum(acc, sel_v[pl.ds(j * 16, 16)])
            m = jnp.max(lax.fori_loop(0, NV + 1, amax,
                                      jnp.full((16,), -2.0, jnp.float32)))

            def amin(j, acc):
                sv = sel_v[pl.ds(j * 16, 16)]
                dv = docs_v[pl.ds(j * 16, 16)]
                return jnp.minimum(acc, jnp.where(sv == m, dv, BIG))
            best = jnp.min(lax.fori_loop(0, NV + 1, amin,
                                         jnp.full((16,), BIG, jnp.int32)))

            def elim(j, _):
                dv = docs_v[pl.ds(j * 16, 16)]
                sv = sel_v[pl.ds(j * 16, 16)]
                sel_v[pl.ds(j * 16, 16)] = jnp.where(
                    dv == best, jnp.float32(-1.0), sv)
                return 0
            lax.fori_loop(0, NV + 1, elim, 0)

            return jnp.where((lane == q) & (pp_vec == r), best, ans_vec)
        return lax.fori_loop(0, ROUNDS, one_round, ans_vec)

    ans_vec = lax.fori_loop(0, QPW, one_query, jnp.zeros((16,), jnp.int32))
    ans_v[...] = ans_vec
    pltpu.sync_copy(ans_v, out_hbm.at[wid])


@jax.jit
def kernel(index_tensor, ranking_tensor, positive_positions, weight):
    idx_flat = index_tensor.reshape(B * E)
    rank_flat = ranking_tensor.reshape(B * E)
    w_exp = jnp.repeat(weight, K)
    pp_pad = jnp.pad(positive_positions, (0, 16))

    run = pl.kernel(
        _body,
        out_type=jax.ShapeDtypeStruct((NW, 16), jnp.int32),
        mesh=plsc.VectorSubcoreMesh(
            core_axis_name="c", subcore_axis_name="s",
            num_cores=NC, num_subcores=NS),
        compiler_params=pltpu.CompilerParams(needs_layout_passes=False),
        scratch_types=[
            pltpu.VMEM((QPW * E,), jnp.int32),
            pltpu.VMEM((QPW * E,), jnp.float32),
            pltpu.VMEM((E,), jnp.float32),
            pltpu.VMEM((16,), jnp.int32),
            pltpu.VMEM((N_DOCS,), jnp.float32),
            pltpu.VMEM((EX,), jnp.float32),
            pltpu.VMEM((EX,), jnp.int32),
            pltpu.VMEM((16,), jnp.int32),
        ],
    )
    out = run(idx_flat, rank_flat, pp_pad, w_exp)
    return out[:, :QPW].reshape(B)


# speedup vs baseline: 424.5106x; 1.7142x over previous
import jax
import jax.numpy as jnp
from jax import lax
from jax.experimental import pallas as pl
from jax.experimental.pallas import tpu as pltpu
from jax.experimental.pallas import tpu_sc as plsc

N_DOCS = 100000
RRF_K = 60.0
B, T, K = 256, 8, 100
E = T * K
NV = E // 16
SCAN = NV + 2
EX = SCAN * 16
NC, NS = 2, 16
NW = NC * NS
QPW = B // NW
BIG = N_DOCS
U0 = 5
U1 = 4


def _body(idx_hbm, rank_hbm, pp_hbm, w_hbm, out_hbm,
          idx_v, rank_v, w_v, pp_v, table_v, docs_v, ans_v):
    wid = lax.axis_index("s") * NC + lax.axis_index("c")
    lane = lax.iota(jnp.int32, 16)
    zero16 = jnp.zeros((16,), jnp.float32)

    pltpu.sync_copy(idx_hbm.at[pl.ds(wid * (QPW * E), QPW * E)], idx_v)
    pltpu.sync_copy(rank_hbm.at[pl.ds(wid * (QPW * E), QPW * E)], rank_v)
    pltpu.sync_copy(w_hbm, w_v)
    pltpu.sync_copy(pp_hbm.at[pl.ds(wid * QPW, 16)], pp_v)

    pp_vec = pp_v[...]
    docs_v[pl.ds(E, 16)] = lane
    docs_v[pl.ds(E + 16, 16)] = lane

    def one_query(q, ans_vec):
        base = q * E

        def p0(j, _):
            for s in range(U0):
                o = (j * U0 + s) * 16
                dv = idx_v[pl.ds(base + o, 16)]
                docs_v[pl.ds(o, 16)] = dv
                plsc.store_scatter(table_v, [dv], zero16)
            return 0
        lax.fori_loop(0, NV // U0, p0, 0)
        plsc.store_scatter(table_v, [lane], zero16)

        def p1(j, _):
            for s in range(U0):
                o = (j * U0 + s) * 16
                dv = docs_v[pl.ds(o, 16)]
                rv = rank_v[pl.ds(base + o, 16)]
                wv = w_v[pl.ds(o, 16)]
                plsc.addupdate_scatter(table_v, [dv], wv / (RRF_K + rv))
            return 0
        lax.fori_loop(0, NV // U0, p1, 0)

        p = jnp.max(jnp.where(lane == q, pp_vec, 0))

        def one_round(r, _):
            def scan(j, carry):
                bsc, bdoc = carry
                for s in range(U1):
                    o = (j * U1 + s) * 16
                    dv = docs_v[pl.ds(o, 16)]
                    sv = plsc.load_gather(table_v, [dv])
                    take = (sv > bsc) | ((sv == bsc) & (dv < bdoc))
                    bsc = jnp.where(take, sv, bsc)
                    bdoc = jnp.where(take, dv, bdoc)
                return bsc, bdoc
            bsc, bdoc = lax.fori_loop(
                0, SCAN // U1, scan,
                (jnp.full((16,), -2.0, jnp.float32),
                 jnp.full((16,), BIG, jnp.int32)))
            m = jnp.max(bsc)
            best = jnp.min(jnp.where(bsc == m, bdoc, BIG))
            plsc.store_scatter(table_v, [jnp.full((16,), best, jnp.int32)],
                               jnp.full((16,), -1.0, jnp.float32),
                               mask=lane == 0)
            return best
        best = lax.fori_loop(0, p + 1, one_round, jnp.int32(0))

        return jnp.where(lane == q, best, ans_vec)

    ans_vec = lax.fori_loop(0, QPW, one_query, jnp.zeros((16,), jnp.int32))
    ans_v[...] = ans_vec
    pltpu.sync_copy(ans_v, out_hbm.at[wid])


@jax.jit
def kernel(index_tensor, ranking_tensor, positive_positions, weight):
    idx_flat = index_tensor.reshape(B * E)
    rank_flat = ranking_tensor.reshape(B * E)
    w_exp = jnp.repeat(weight, K)
    pp_pad = jnp.pad(positive_positions, (0, 16))

    run = pl.kernel(
        _body,
        out_type=jax.ShapeDtypeStruct((NW, 16), jnp.int32),
        mesh=plsc.VectorSubcoreMesh(
            core_axis_name="c", subcore_axis_name="s",
            num_cores=NC, num_subcores=NS),
        compiler_params=pltpu.CompilerParams(needs_layout_passes=False),
        scratch_types=[
            pltpu.VMEM((QPW * E,), jnp.int32),
            pltpu.VMEM((QPW * E,), jnp.float32),
            pltpu.VMEM((E,), jnp.float32),
            pltpu.VMEM((16,), jnp.int32),
            pltpu.VMEM((N_DOCS,), jnp.float32),
            pltpu.VMEM((EX,), jnp.int32),
            pltpu.VMEM((16,), jnp.int32),
        ],
    )
    out = run(idx_flat, rank_flat, pp_pad, w_exp)
    return out[:, :QPW].reshape(B)


# speedup vs baseline: 438.2936x; 1.0325x over previous
import jax
import jax.numpy as jnp
from jax import lax
from jax.experimental import pallas as pl
from jax.experimental.pallas import tpu as pltpu
from jax.experimental.pallas import tpu_sc as plsc

N_DOCS = 100000
RRF_K = 60.0
B, T, K = 256, 8, 100
E = T * K
NV = E // 16
SCAN = NV + 2
EX = SCAN * 16
NC, NS = 2, 16
NW = NC * NS
QPW = B // NW
BIG = N_DOCS


def _t_vec(j, lane):
    t_base = (j * 16) // K
    boundary = (t_base + 1) * K
    return jnp.where(lane + j * 16 >= boundary, t_base + 1, t_base)


def _body(idx_hbm, rank_hbm, pp_hbm, w_hbm, out_hbm,
          idx_v, rank_v, w_v, pp_v, table_v, docs_v, ans_v):
    wid = lax.axis_index("s") * NC + lax.axis_index("c")
    lane = lax.iota(jnp.int32, 16)
    zero16 = jnp.zeros((16,), jnp.float32)

    pltpu.sync_copy(idx_hbm.at[pl.ds(wid * (QPW * E), QPW * E)], idx_v)
    pltpu.sync_copy(rank_hbm.at[pl.ds(wid * (QPW * E), QPW * E)], rank_v)
    pltpu.sync_copy(w_hbm, w_v)
    pp_off = jnp.where(wid == NW - 1, 8, 0)
    pltpu.sync_copy(pp_hbm.at[pl.ds(wid * QPW - pp_off, 16)], pp_v)

    pp_vec = pp_v[...]
    docs_v[pl.ds(E, 16)] = lane
    docs_v[pl.ds(E + 16, 16)] = lane

    def one_query(q, ans_vec):
        base = q * E

        for j in range(NV):
            dv = idx_v[pl.ds(base + j * 16, 16)]
            docs_v[pl.ds(j * 16, 16)] = dv
            plsc.store_scatter(table_v, [dv], zero16)
        plsc.store_scatter(table_v, [lane], zero16)

        for j in range(NV):
            dv = docs_v[pl.ds(j * 16, 16)]
            rv = rank_v[pl.ds(base + j * 16, 16)]
            wv = w_v[pl.ds(j * 16, 16)]
            plsc.addupdate_scatter(table_v, [dv], wv / (RRF_K + rv))

        p = jnp.max(jnp.where(lane == q + pp_off, pp_vec, 0))

        def one_round(r, _):
            bsc = [jnp.full((16,), -2.0, jnp.float32) for _ in range(4)]
            bdoc = [jnp.full((16,), BIG, jnp.int32) for _ in range(4)]
            for j in range(SCAN):
                c = j % 4
                dv = docs_v[pl.ds(j * 16, 16)]
                sv = plsc.load_gather(table_v, [dv])
                take = (sv > bsc[c]) | ((sv == bsc[c]) & (dv < bdoc[c]))
                bsc[c] = jnp.where(take, sv, bsc[c])
                bdoc[c] = jnp.where(take, dv, bdoc[c])
            for c in (1, 2, 3):
                take = (bsc[c] > bsc[0]) | ((bsc[c] == bsc[0])
                                            & (bdoc[c] < bdoc[0]))
                bsc[0] = jnp.where(take, bsc[c], bsc[0])
                bdoc[0] = jnp.where(take, bdoc[c], bdoc[0])
            m = jnp.max(bsc[0])
            best = jnp.min(jnp.where(bsc[0] == m, bdoc[0], BIG))
            plsc.store_scatter(table_v, [jnp.full((16,), best, jnp.int32)],
                               jnp.full((16,), -1.0, jnp.float32),
                               mask=lane == 0)
            return best
        best = lax.fori_loop(0, p + 1, one_round, jnp.int32(0))

        return jnp.where(lane == q, best, ans_vec)

    ans_vec = lax.fori_loop(0, QPW, one_query, jnp.zeros((16,), jnp.int32))
    ans_v[...] = ans_vec
    pltpu.sync_copy(ans_v, out_hbm.at[wid])


@jax.jit
def kernel(index_tensor, ranking_tensor, positive_positions, weight):
    idx_flat = index_tensor.reshape(B * E)
    rank_flat = ranking_tensor.reshape(B * E)
    w_exp = jnp.repeat(weight, K)

    run = pl.kernel(
        _body,
        out_type=jax.ShapeDtypeStruct((NW, 16), jnp.int32),
        mesh=plsc.VectorSubcoreMesh(
            core_axis_name="c", subcore_axis_name="s",
            num_cores=NC, num_subcores=NS),
        compiler_params=pltpu.CompilerParams(needs_layout_passes=False),
        scratch_types=[
            pltpu.VMEM((QPW * E,), jnp.int32),
            pltpu.VMEM((QPW * E,), jnp.float32),
            pltpu.VMEM((E,), jnp.float32),
            pltpu.VMEM((16,), jnp.int32),
            pltpu.VMEM((N_DOCS,), jnp.float32),
            pltpu.VMEM((EX,), jnp.int32),
            pltpu.VMEM((16,), jnp.int32),
        ],
    )
    out = run(idx_flat, rank_flat, positive_positions, w_exp)
    return out[:, :QPW].reshape(B)


# speedup vs baseline: 459.3602x; 1.0481x over previous
import jax
import jax.numpy as jnp
from jax import lax
from jax.experimental import pallas as pl
from jax.experimental.pallas import tpu as pltpu
from jax.experimental.pallas import tpu_sc as plsc

N_DOCS = 100000
RRF_K = 60.0
B, T, K = 256, 8, 100
E = T * K
NV = E // 16
SCAN = NV + 2
NC, NS = 2, 16
NW = NC * NS
QPW = B // NW
BIG = N_DOCS


def _tk_vecs(j, lane):
    t_base = (j * 16) // K
    boundary = (t_base + 1) * K
    o_vec = lane + j * 16
    cross = o_vec >= boundary
    t_vec = jnp.where(cross, t_base + 1, t_base)
    k_vec = jnp.where(cross, o_vec - (t_base + 1) * K, o_vec - t_base * K)
    return t_vec, k_vec


def _body(idx_hbm, rank_hbm, pp_hbm, w_hbm, out_hbm,
          idx_v, rank_v, w_v, pp_v, table_v, ans_v):
    wid = lax.axis_index("s") * NC + lax.axis_index("c")
    lane = lax.iota(jnp.int32, 16)
    zero16 = jnp.zeros((16,), jnp.float32)

    pltpu.sync_copy(idx_hbm.at[pl.ds(wid * QPW, QPW)], idx_v)
    pltpu.sync_copy(rank_hbm.at[pl.ds(wid * QPW, QPW)], rank_v)
    pltpu.sync_copy(w_hbm, w_v)
    pp_off = jnp.where(wid == NW - 1, 8, 0)
    pltpu.sync_copy(pp_hbm.at[pl.ds(wid * QPW - pp_off, 16)], pp_v)

    pp_vec = pp_v[...]

    def entry_docs(q16, j, lane):
        if j >= NV:
            return lane
        t_vec, k_vec = _tk_vecs(j, lane)
        return plsc.load_gather(idx_v, [q16, t_vec, k_vec])

    def one_query(q, ans_vec):
        q16 = jnp.broadcast_to(q, (16,))

        for j in range(NV):
            plsc.store_scatter(table_v, [entry_docs(q16, j, lane)], zero16)
        plsc.store_scatter(table_v, [lane], zero16)

        for j in range(NV):
            t_vec, k_vec = _tk_vecs(j, lane)
            dv = plsc.load_gather(idx_v, [q16, t_vec, k_vec])
            rv = plsc.load_gather(rank_v, [q16, t_vec, k_vec])
            wv = w_v[pl.ds(j * 16, 16)]
            plsc.addupdate_scatter(table_v, [dv], wv / (RRF_K + rv))

        p = jnp.max(jnp.where(lane == q + pp_off, pp_vec, 0))

        def one_round(r, _):
            bsc = [jnp.full((16,), -2.0, jnp.float32) for _ in range(4)]
            bdoc = [jnp.full((16,), BIG, jnp.int32) for _ in range(4)]
            for j in range(SCAN):
                c = j % 4
                dv = entry_docs(q16, j, lane)
                sv = plsc.load_gather(table_v, [dv])
                take = (sv > bsc[c]) | ((sv == bsc[c]) & (dv < bdoc[c]))
                bsc[c] = jnp.where(take, sv, bsc[c])
                bdoc[c] = jnp.where(take, dv, bdoc[c])
            for c in (1, 2, 3):
                take = (bsc[c] > bsc[0]) | ((bsc[c] == bsc[0])
                                            & (bdoc[c] < bdoc[0]))
                bsc[0] = jnp.where(take, bsc[c], bsc[0])
                bdoc[0] = jnp.where(take, bdoc[c], bdoc[0])
            m = jnp.max(bsc[0])
            best = jnp.min(jnp.where(bsc[0] == m, bdoc[0], BIG))
            plsc.store_scatter(table_v, [jnp.full((16,), best, jnp.int32)],
                               jnp.full((16,), -1.0, jnp.float32),
                               mask=lane == 0)
            return best
        best = lax.fori_loop(0, p + 1, one_round, jnp.int32(0))

        return jnp.where(lane == q, best, ans_vec)

    ans_vec = lax.fori_loop(0, QPW, one_query, jnp.zeros((16,), jnp.int32))
    ans_v[...] = ans_vec
    pltpu.sync_copy(ans_v, out_hbm.at[wid])


@jax.jit
def kernel(index_tensor, ranking_tensor, positive_positions, weight):
    w_exp = jnp.repeat(weight, K)

    run = pl.kernel(
        _body,
        out_type=jax.ShapeDtypeStruct((NW, 16), jnp.int32),
        mesh=plsc.VectorSubcoreMesh(
            core_axis_name="c", subcore_axis_name="s",
            num_cores=NC, num_subcores=NS),
        compiler_params=pltpu.CompilerParams(needs_layout_passes=False),
        scratch_types=[
            pltpu.VMEM((QPW, T, K), jnp.int32),
            pltpu.VMEM((QPW, T, K), jnp.float32),
            pltpu.VMEM((E,), jnp.float32),
            pltpu.VMEM((16,), jnp.int32),
            pltpu.VMEM((N_DOCS,), jnp.float32),
            pltpu.VMEM((16,), jnp.int32),
        ],
    )
    out = run(index_tensor, ranking_tensor, positive_positions, w_exp)
    return out[:, :QPW].reshape(B)
